# baseline (speedup 1.0000x reference)
import functools

import jax
import jax.numpy as jnp
from jax import lax
from jax.experimental import pallas as pl
from jax.experimental.pallas import tpu as pltpu
from jax.experimental.pallas import tpu_sc as plsc

N = 10000
D = 128
NP = 10240
K = 128
NCHUNK = 160
GCH = 32
NBUF = 2
EPAD = 16 * NCHUNK * K
RPT = NP // 16

def _deg_body(col_ref, deg_out, deg_sh, colbuf, ones, zbuf):
    c = lax.axis_index("c")
    s = lax.axis_index("s")

    def _zero(i, _):
        zbuf[pl.ds(i * 16, 16)] = jnp.zeros((16,), jnp.float32)
        return 0

    lax.fori_loop(0, RPT // 16, _zero, 0)

    def _one(i, _):
        ones[pl.ds(i * 16, 16)] = jnp.full((16,), 1.0, jnp.float32)
        return 0

    lax.fori_loop(0, K // 16, _one, 0)

    pltpu.sync_copy(zbuf, deg_sh.at[pl.ds(s * RPT, RPT)])
    pltpu.sync_copy(col_ref.at[s, pl.ds(c * (NCHUNK // 2), NCHUNK // 2)],
                    colbuf)
    plsc.subcore_barrier()

    def _scat(j, _):
        pltpu.sync_copy(ones, deg_sh.at[colbuf.at[j]], add=True)
        return 0

    lax.fori_loop(0, NCHUNK // 2, _scat, 0)
    plsc.subcore_barrier()
    pltpu.sync_copy(deg_sh.at[pl.ds(s * RPT, RPT)],
                    deg_out.at[c, pl.ds(s * RPT, RPT)])


@functools.cache
def _deg_call():
    return pl.kernel(
        _deg_body,
        out_type=jax.ShapeDtypeStruct((2, NP), jnp.float32),
        mesh=plsc.VectorSubcoreMesh(core_axis_name="c", subcore_axis_name="s"),
        scratch_types=[
            pltpu.VMEM_SHARED((NP,), jnp.float32),
            pltpu.VMEM((NCHUNK // 2, K), jnp.int32),
            pltpu.VMEM((K,), jnp.float32),
            pltpu.VMEM((RPT,), jnp.float32),
        ],
    )


_BR = 512


def _dense_body(x_ref, wm_ref, wv_ref, bm_ref, bv_ref, deg_ref,
                feats_ref, dinv_ref):
    x = x_ref[...]
    mean = jnp.maximum(
        jnp.dot(x, wm_ref[...], preferred_element_type=jnp.float32)
        + bm_ref[...], 0.0)
    var = jnp.maximum(
        jnp.dot(x, wv_ref[...], preferred_element_type=jnp.float32)
        + bv_ref[...], 0.0)
    att = jnp.exp(-var)
    mean = mean * att
    var = var * (att * att)
    deg = deg_ref[0] + deg_ref[1] + 1.0
    dinv = lax.rsqrt(deg)
    feats_ref[0, :, :] = mean * dinv
    feats_ref[1, :, :] = var * (dinv * dinv)
    dinv_ref[...] = dinv


_dense_call = pl.pallas_call(
    _dense_body,
    grid=(NP // _BR,),
    in_specs=[
        pl.BlockSpec((_BR, D), lambda i: (i, 0)),
        pl.BlockSpec((D, D), lambda i: (0, 0)),
        pl.BlockSpec((D, D), lambda i: (0, 0)),
        pl.BlockSpec((1, D), lambda i: (0, 0)),
        pl.BlockSpec((1, D), lambda i: (0, 0)),
        pl.BlockSpec((2, _BR, 1), lambda i: (0, i, 0)),
    ],
    out_specs=[
        pl.BlockSpec((2, _BR, D), lambda i: (0, i, 0)),
        pl.BlockSpec((_BR, 1), lambda i: (i, 0)),
    ],
    out_shape=[
        jax.ShapeDtypeStruct((2, NP, D), jnp.float32),
        jax.ShapeDtypeStruct((NP, 1), jnp.float32),
    ],
)


def _spmm_body(grow_ref, col_ref, flat_ref, out_ref,
               acc_sh, rowbuf, colbuf, fbufs, gsems, ssems):
    c = lax.axis_index("c")
    s = lax.axis_index("s")
    r0 = s * RPT
    pltpu.sync_copy(flat_ref.at[pl.ds(c * NP + r0, RPT)],
                    acc_sh.at[pl.ds(r0, RPT)])
    plsc.subcore_barrier()

    def _gather(j, b):
        pltpu.async_copy(flat_ref.at[rowbuf.at[j]], fbufs[b], gsems[b])

    def _gwait(b):
        pltpu.make_async_copy(flat_ref.at[pl.ds(0, K)], fbufs[b],
                              gsems[b]).wait()

    def _scatter(j, b):
        pltpu.async_copy(fbufs[b], acc_sh.at[colbuf.at[j]], ssems[b],
                         add=True)

    def _swait(b):
        pltpu.make_async_copy(fbufs[b], acc_sh.at[colbuf.at[0]],
                              ssems[b]).wait()

    def _stage(g, _):
        pltpu.sync_copy(grow_ref.at[c, s, pl.ds(g * GCH, GCH)], rowbuf)
        pltpu.sync_copy(col_ref.at[s, pl.ds(g * GCH, GCH)], colbuf)
        for b in range(NBUF):
            _gather(b, b)

        def _quad(q, _):
            j = NBUF * q
            for b in range(NBUF):
                _gwait(b)
                _scatter(j + b, b)
            for b in range(NBUF):
                _swait(b)

                @pl.when(j + NBUF + b < GCH)
                def _():
                    _gather(j + NBUF + b, b)
            return 0

        lax.fori_loop(0, GCH // NBUF, _quad, 0)
        return 0

    lax.fori_loop(0, NCHUNK // GCH, _stage, 0)
    plsc.subcore_barrier()
    pltpu.sync_copy(acc_sh.at[pl.ds(r0, RPT)], out_ref.at[c, pl.ds(r0, RPT)])


@functools.cache
def _spmm_call():
    return pl.kernel(
        _spmm_body,
        out_type=jax.ShapeDtypeStruct((2, NP, D), jnp.float32),
        mesh=plsc.VectorSubcoreMesh(core_axis_name="c", subcore_axis_name="s"),
        scratch_types=[
            pltpu.VMEM_SHARED((NP, D), jnp.float32),
            pltpu.VMEM((GCH, K), jnp.int32),
            pltpu.VMEM((GCH, K), jnp.int32),
            [pltpu.VMEM((K, D), jnp.float32)] * NBUF,
            [pltpu.SemaphoreType.DMA] * NBUF,
            [pltpu.SemaphoreType.DMA] * NBUF,
        ],
    )


_BD = 1000


def _scale_body(acc_ref, dinv_ref, mo_ref, vo_ref):
    dinv = dinv_ref[...]
    mo_ref[...] = acc_ref[0] * dinv
    vo_ref[...] = acc_ref[1] * (dinv * dinv)


_scale_call = pl.pallas_call(
    _scale_body,
    grid=(N // _BD,),
    in_specs=[
        pl.BlockSpec((2, _BD, D), lambda i: (0, i, 0)),
        pl.BlockSpec((_BD, 1), lambda i: (i, 0)),
    ],
    out_specs=[
        pl.BlockSpec((_BD, D), lambda i: (i, 0)),
        pl.BlockSpec((_BD, D), lambda i: (i, 0)),
    ],
    out_shape=[
        jax.ShapeDtypeStruct((N, D), jnp.float32),
        jax.ShapeDtypeStruct((N, D), jnp.float32),
    ],
)


def kernel(x, edge_index, W_mean, W_var, bias_mean, bias_var):
    row = edge_index[0]
    col = edge_index[1]
    pad = EPAD - row.shape[0]
    row_p = jnp.concatenate([row, jnp.zeros((pad,), jnp.int32)])
    col_p = jnp.concatenate([col, jnp.full((pad,), N, jnp.int32)])
    col_r = col_p.reshape(16, NCHUNK, K)
    grow = jnp.concatenate([row_p, row_p + NP]).reshape(2, 16, NCHUNK, K)

    degp = _deg_call()(col_r)

    x_p = jnp.pad(x, ((0, NP - N), (0, 0)))
    feats2, dinv = _dense_call(
        x_p, W_mean, W_var,
        bias_mean.reshape(1, D), bias_var.reshape(1, D),
        degp.reshape(2, NP, 1))

    flat = feats2.reshape(2 * NP, D)
    acc = _spmm_call()(grow, col_r, flat)

    mean_out, var_out = _scale_call(acc, dinv)
    return (mean_out, var_out)


# speedup vs baseline: 1.1293x; 1.1293x over previous
import functools

import jax
import jax.numpy as jnp
from jax import lax
from jax.experimental import pallas as pl
from jax.experimental.pallas import tpu as pltpu
from jax.experimental.pallas import tpu_sc as plsc

N = 10000
D = 128
NP = 10240
K = 64
NCHUNK = 320
GCH = 64
NBUF = 4
EPAD = 16 * NCHUNK * K
RPT = NP // 16

def _deg_body(col_ref, deg_out, deg_sh, colbuf, ones, zbuf):
    c = lax.axis_index("c")
    s = lax.axis_index("s")

    def _zero(i, _):
        zbuf[pl.ds(i * 16, 16)] = jnp.zeros((16,), jnp.float32)
        return 0

    lax.fori_loop(0, RPT // 16, _zero, 0)

    def _one(i, _):
        ones[pl.ds(i * 16, 16)] = jnp.full((16,), 1.0, jnp.float32)
        return 0

    lax.fori_loop(0, K // 16, _one, 0)

    pltpu.sync_copy(zbuf, deg_sh.at[pl.ds(s * RPT, RPT)])
    pltpu.sync_copy(col_ref.at[s, pl.ds(c * (NCHUNK // 2), NCHUNK // 2)],
                    colbuf)
    plsc.subcore_barrier()

    def _scat(j, _):
        pltpu.sync_copy(ones, deg_sh.at[colbuf.at[j]], add=True)
        return 0

    lax.fori_loop(0, NCHUNK // 2, _scat, 0)
    plsc.subcore_barrier()
    pltpu.sync_copy(deg_sh.at[pl.ds(s * RPT, RPT)],
                    deg_out.at[c, pl.ds(s * RPT, RPT)])


@functools.cache
def _deg_call():
    return pl.kernel(
        _deg_body,
        out_type=jax.ShapeDtypeStruct((2, NP), jnp.float32),
        mesh=plsc.VectorSubcoreMesh(core_axis_name="c", subcore_axis_name="s"),
        scratch_types=[
            pltpu.VMEM_SHARED((NP,), jnp.float32),
            pltpu.VMEM((NCHUNK // 2, K), jnp.int32),
            pltpu.VMEM((K,), jnp.float32),
            pltpu.VMEM((RPT,), jnp.float32),
        ],
    )


_BR = 512


def _dense_body(x_ref, wm_ref, wv_ref, bm_ref, bv_ref, deg_ref,
                feats_ref, dinv_ref):
    x = x_ref[...]
    mean = jnp.maximum(
        jnp.dot(x, wm_ref[...], preferred_element_type=jnp.float32)
        + bm_ref[...], 0.0)
    var = jnp.maximum(
        jnp.dot(x, wv_ref[...], preferred_element_type=jnp.float32)
        + bv_ref[...], 0.0)
    att = jnp.exp(-var)
    mean = mean * att
    var = var * (att * att)
    deg = deg_ref[0] + deg_ref[1] + 1.0
    dinv = lax.rsqrt(deg)
    feats_ref[0, :, :] = mean * dinv
    feats_ref[1, :, :] = var * (dinv * dinv)
    dinv_ref[...] = dinv


_dense_call = pl.pallas_call(
    _dense_body,
    grid=(NP // _BR,),
    in_specs=[
        pl.BlockSpec((_BR, D), lambda i: (i, 0)),
        pl.BlockSpec((D, D), lambda i: (0, 0)),
        pl.BlockSpec((D, D), lambda i: (0, 0)),
        pl.BlockSpec((1, D), lambda i: (0, 0)),
        pl.BlockSpec((1, D), lambda i: (0, 0)),
        pl.BlockSpec((2, _BR, 1), lambda i: (0, i, 0)),
    ],
    out_specs=[
        pl.BlockSpec((2, _BR, D), lambda i: (0, i, 0)),
        pl.BlockSpec((_BR, 1), lambda i: (i, 0)),
    ],
    out_shape=[
        jax.ShapeDtypeStruct((2, NP, D), jnp.float32),
        jax.ShapeDtypeStruct((NP, 1), jnp.float32),
    ],
)


def _spmm_body(grow_ref, col_ref, flat_ref, out_ref,
               acc_sh, rowbuf, colbuf, fbufs, gsems, ssems):
    c = lax.axis_index("c")
    s = lax.axis_index("s")
    r0 = s * RPT
    pltpu.sync_copy(flat_ref.at[pl.ds(c * NP + r0, RPT)],
                    acc_sh.at[pl.ds(r0, RPT)])
    plsc.subcore_barrier()

    def _gather(j, b):
        pltpu.async_copy(flat_ref.at[rowbuf.at[j]], fbufs[b], gsems[b])

    def _gwait(b):
        pltpu.make_async_copy(flat_ref.at[pl.ds(0, K)], fbufs[b],
                              gsems[b]).wait()

    def _scatter(j, b):
        pltpu.async_copy(fbufs[b], acc_sh.at[colbuf.at[j]], ssems[b],
                         add=True)

    def _swait(b):
        pltpu.make_async_copy(fbufs[b], acc_sh.at[colbuf.at[0]],
                              ssems[b]).wait()

    def _stage(g, _):
        pltpu.sync_copy(grow_ref.at[c, s, pl.ds(g * GCH, GCH)], rowbuf)
        pltpu.sync_copy(col_ref.at[s, pl.ds(g * GCH, GCH)], colbuf)
        for b in range(NBUF):
            _gather(b, b)

        def _quad(q, _):
            j = NBUF * q
            for b in range(NBUF):
                _gwait(b)
                _scatter(j + b, b)
            for b in range(NBUF):
                _swait(b)

                @pl.when(j + NBUF + b < GCH)
                def _():
                    _gather(j + NBUF + b, b)
            return 0

        lax.fori_loop(0, GCH // NBUF, _quad, 0)
        return 0

    lax.fori_loop(0, NCHUNK // GCH, _stage, 0)
    plsc.subcore_barrier()
    pltpu.sync_copy(acc_sh.at[pl.ds(r0, RPT)], out_ref.at[c, pl.ds(r0, RPT)])


@functools.cache
def _spmm_call():
    return pl.kernel(
        _spmm_body,
        out_type=jax.ShapeDtypeStruct((2, NP, D), jnp.float32),
        mesh=plsc.VectorSubcoreMesh(core_axis_name="c", subcore_axis_name="s"),
        scratch_types=[
            pltpu.VMEM_SHARED((NP, D), jnp.float32),
            pltpu.VMEM((GCH, K), jnp.int32),
            pltpu.VMEM((GCH, K), jnp.int32),
            [pltpu.VMEM((K, D), jnp.float32)] * NBUF,
            [pltpu.SemaphoreType.DMA] * NBUF,
            [pltpu.SemaphoreType.DMA] * NBUF,
        ],
    )


_BD = 1000


def _scale_body(acc_ref, dinv_ref, mo_ref, vo_ref):
    dinv = dinv_ref[...]
    mo_ref[...] = acc_ref[0] * dinv
    vo_ref[...] = acc_ref[1] * (dinv * dinv)


_scale_call = pl.pallas_call(
    _scale_body,
    grid=(N // _BD,),
    in_specs=[
        pl.BlockSpec((2, _BD, D), lambda i: (0, i, 0)),
        pl.BlockSpec((_BD, 1), lambda i: (i, 0)),
    ],
    out_specs=[
        pl.BlockSpec((_BD, D), lambda i: (i, 0)),
        pl.BlockSpec((_BD, D), lambda i: (i, 0)),
    ],
    out_shape=[
        jax.ShapeDtypeStruct((N, D), jnp.float32),
        jax.ShapeDtypeStruct((N, D), jnp.float32),
    ],
)


def kernel(x, edge_index, W_mean, W_var, bias_mean, bias_var):
    row = edge_index[0]
    col = edge_index[1]
    pad = EPAD - row.shape[0]
    row_p = jnp.concatenate([row, jnp.zeros((pad,), jnp.int32)])
    col_p = jnp.concatenate([col, jnp.full((pad,), N, jnp.int32)])
    col_r = col_p.reshape(16, NCHUNK, K)
    grow = jnp.concatenate([row_p, row_p + NP]).reshape(2, 16, NCHUNK, K)

    degp = _deg_call()(col_r)

    x_p = jnp.pad(x, ((0, NP - N), (0, 0)))
    feats2, dinv = _dense_call(
        x_p, W_mean, W_var,
        bias_mean.reshape(1, D), bias_var.reshape(1, D),
        degp.reshape(2, NP, 1))

    flat = feats2.reshape(2 * NP, D)
    acc = _spmm_call()(grow, col_r, flat)

    mean_out, var_out = _scale_call(acc, dinv)
    return (mean_out, var_out)


# speedup vs baseline: 1.1402x; 1.0096x over previous
import functools

import jax
import jax.numpy as jnp
from jax import lax
from jax.experimental import pallas as pl
from jax.experimental.pallas import tpu as pltpu
from jax.experimental.pallas import tpu_sc as plsc

N = 10000
D = 128
NP = 10240
K = 32
NCHUNK = 640
GCH = 64
NBUF = 8
EPAD = 16 * NCHUNK * K
RPT = NP // 16

def _deg_body(col_ref, deg_out, deg_sh, colbuf, ones, zbuf):
    c = lax.axis_index("c")
    s = lax.axis_index("s")

    def _zero(i, _):
        zbuf[pl.ds(i * 16, 16)] = jnp.zeros((16,), jnp.float32)
        return 0

    lax.fori_loop(0, RPT // 16, _zero, 0)

    def _one(i, _):
        ones[pl.ds(i * 16, 16)] = jnp.full((16,), 1.0, jnp.float32)
        return 0

    lax.fori_loop(0, K // 16, _one, 0)

    pltpu.sync_copy(zbuf, deg_sh.at[pl.ds(s * RPT, RPT)])
    pltpu.sync_copy(col_ref.at[s, pl.ds(c * (NCHUNK // 2), NCHUNK // 2)],
                    colbuf)
    plsc.subcore_barrier()

    def _scat(j, _):
        pltpu.sync_copy(ones, deg_sh.at[colbuf.at[j]], add=True)
        return 0

    lax.fori_loop(0, NCHUNK // 2, _scat, 0)
    plsc.subcore_barrier()
    pltpu.sync_copy(deg_sh.at[pl.ds(s * RPT, RPT)],
                    deg_out.at[c, pl.ds(s * RPT, RPT)])


@functools.cache
def _deg_call():
    return pl.kernel(
        _deg_body,
        out_type=jax.ShapeDtypeStruct((2, NP), jnp.float32),
        mesh=plsc.VectorSubcoreMesh(core_axis_name="c", subcore_axis_name="s"),
        scratch_types=[
            pltpu.VMEM_SHARED((NP,), jnp.float32),
            pltpu.VMEM((NCHUNK // 2, K), jnp.int32),
            pltpu.VMEM((K,), jnp.float32),
            pltpu.VMEM((RPT,), jnp.float32),
        ],
    )


_BR = 512


def _dense_body(x_ref, wm_ref, wv_ref, bm_ref, bv_ref, deg_ref,
                feats_ref, dinv_ref):
    x = x_ref[...]
    mean = jnp.maximum(
        jnp.dot(x, wm_ref[...], preferred_element_type=jnp.float32)
        + bm_ref[...], 0.0)
    var = jnp.maximum(
        jnp.dot(x, wv_ref[...], preferred_element_type=jnp.float32)
        + bv_ref[...], 0.0)
    att = jnp.exp(-var)
    mean = mean * att
    var = var * (att * att)
    deg = deg_ref[0] + deg_ref[1] + 1.0
    dinv = lax.rsqrt(deg)
    feats_ref[0, :, :] = mean * dinv
    feats_ref[1, :, :] = var * (dinv * dinv)
    dinv_ref[...] = dinv


_dense_call = pl.pallas_call(
    _dense_body,
    grid=(NP // _BR,),
    in_specs=[
        pl.BlockSpec((_BR, D), lambda i: (i, 0)),
        pl.BlockSpec((D, D), lambda i: (0, 0)),
        pl.BlockSpec((D, D), lambda i: (0, 0)),
        pl.BlockSpec((1, D), lambda i: (0, 0)),
        pl.BlockSpec((1, D), lambda i: (0, 0)),
        pl.BlockSpec((2, _BR, 1), lambda i: (0, i, 0)),
    ],
    out_specs=[
        pl.BlockSpec((2, _BR, D), lambda i: (0, i, 0)),
        pl.BlockSpec((_BR, 1), lambda i: (i, 0)),
    ],
    out_shape=[
        jax.ShapeDtypeStruct((2, NP, D), jnp.float32),
        jax.ShapeDtypeStruct((NP, 1), jnp.float32),
    ],
)


def _spmm_body(grow_ref, col_ref, flat_ref, out_ref,
               acc_sh, rowbuf, colbuf, fbufs, gsems, ssems):
    c = lax.axis_index("c")
    s = lax.axis_index("s")
    r0 = s * RPT
    pltpu.sync_copy(flat_ref.at[pl.ds(c * NP + r0, RPT)],
                    acc_sh.at[pl.ds(r0, RPT)])
    plsc.subcore_barrier()

    def _gather(j, b):
        pltpu.async_copy(flat_ref.at[rowbuf.at[j]], fbufs[b], gsems[b])

    def _gwait(b):
        pltpu.make_async_copy(flat_ref.at[pl.ds(0, K)], fbufs[b],
                              gsems[b]).wait()

    def _scatter(j, b):
        pltpu.async_copy(fbufs[b], acc_sh.at[colbuf.at[j]], ssems[b],
                         add=True)

    def _swait(b):
        pltpu.make_async_copy(fbufs[b], acc_sh.at[colbuf.at[0]],
                              ssems[b]).wait()

    def _stage(g, _):
        pltpu.sync_copy(grow_ref.at[c, s, pl.ds(g * GCH, GCH)], rowbuf)
        pltpu.sync_copy(col_ref.at[s, pl.ds(g * GCH, GCH)], colbuf)
        for b in range(NBUF):
            _gather(b, b)

        def _quad(q, _):
            j = NBUF * q
            for b in range(NBUF):
                _gwait(b)
                _scatter(j + b, b)
            for b in range(NBUF):
                _swait(b)

                @pl.when(j + NBUF + b < GCH)
                def _():
                    _gather(j + NBUF + b, b)
            return 0

        lax.fori_loop(0, GCH // NBUF, _quad, 0)
        return 0

    lax.fori_loop(0, NCHUNK // GCH, _stage, 0)
    plsc.subcore_barrier()
    pltpu.sync_copy(acc_sh.at[pl.ds(r0, RPT)], out_ref.at[c, pl.ds(r0, RPT)])


@functools.cache
def _spmm_call():
    return pl.kernel(
        _spmm_body,
        out_type=jax.ShapeDtypeStruct((2, NP, D), jnp.float32),
        mesh=plsc.VectorSubcoreMesh(core_axis_name="c", subcore_axis_name="s"),
        scratch_types=[
            pltpu.VMEM_SHARED((NP, D), jnp.float32),
            pltpu.VMEM((GCH, K), jnp.int32),
            pltpu.VMEM((GCH, K), jnp.int32),
            [pltpu.VMEM((K, D), jnp.float32)] * NBUF,
            [pltpu.SemaphoreType.DMA] * NBUF,
            [pltpu.SemaphoreType.DMA] * NBUF,
        ],
    )


_BD = 1000


def _scale_body(acc_ref, dinv_ref, mo_ref, vo_ref):
    dinv = dinv_ref[...]
    mo_ref[...] = acc_ref[0] * dinv
    vo_ref[...] = acc_ref[1] * (dinv * dinv)


_scale_call = pl.pallas_call(
    _scale_body,
    grid=(N // _BD,),
    in_specs=[
        pl.BlockSpec((2, _BD, D), lambda i: (0, i, 0)),
        pl.BlockSpec((_BD, 1), lambda i: (i, 0)),
    ],
    out_specs=[
        pl.BlockSpec((_BD, D), lambda i: (i, 0)),
        pl.BlockSpec((_BD, D), lambda i: (i, 0)),
    ],
    out_shape=[
        jax.ShapeDtypeStruct((N, D), jnp.float32),
        jax.ShapeDtypeStruct((N, D), jnp.float32),
    ],
)


def kernel(x, edge_index, W_mean, W_var, bias_mean, bias_var):
    row = edge_index[0]
    col = edge_index[1]
    pad = EPAD - row.shape[0]
    row_p = jnp.concatenate([row, jnp.zeros((pad,), jnp.int32)])
    col_p = jnp.concatenate([col, jnp.full((pad,), N, jnp.int32)])
    col_r = col_p.reshape(16, NCHUNK, K)
    grow = jnp.concatenate([row_p, row_p + NP]).reshape(2, 16, NCHUNK, K)

    degp = _deg_call()(col_r)

    x_p = jnp.pad(x, ((0, NP - N), (0, 0)))
    feats2, dinv = _dense_call(
        x_p, W_mean, W_var,
        bias_mean.reshape(1, D), bias_var.reshape(1, D),
        degp.reshape(2, NP, 1))

    flat = feats2.reshape(2 * NP, D)
    acc = _spmm_call()(grow, col_r, flat)

    mean_out, var_out = _scale_call(acc, dinv)
    return (mean_out, var_out)
